# Initial kernel scaffold; baseline (speedup 1.0000x reference)
import jax
import jax.numpy as jnp
from jax.experimental import pallas as pl


def kernel(x, edge_index, edge_weight, layer1, layer2, mask1, mask2):
    raise NotImplementedError("write your pallas kernel here")


# speedup vs baseline: 8.9578x; 8.9578x over previous
import functools

import jax
import jax.numpy as jnp
from jax import lax
from jax.experimental import pallas as pl
from jax.experimental.pallas import tpu as pltpu
from jax.experimental.pallas import tpu_sc as plsc

T = 4

_NC = 2
_NS = 16
_LANES = 16
_NW = _NC * _NS
_K = 80


def _seg_sum_sc(vals, src, dst, w):
    n_nodes, d = vals.shape
    e_total = src.shape[0]
    ept = e_total // _NW
    nchunk = ept // _K
    rpt_main = (-(-n_nodes // _NS) + 7) // 8 * 8
    rpt_last = n_nodes - (_NS - 1) * rpt_main
    assert ept * _NW == e_total and nchunk * _K == ept
    assert rpt_last > 0 and rpt_last % 8 == 0 and d % _LANES == 0

    mesh = plsc.VectorSubcoreMesh(core_axis_name="c", subcore_axis_name="s")

    @functools.partial(
        pl.kernel,
        out_type=jax.ShapeDtypeStruct((_NC * n_nodes, d), jnp.float32),
        mesh=mesh,
        scratch_types=[
            pltpu.VMEM_SHARED((n_nodes, d), jnp.float32),
            pltpu.VMEM((_K,), jnp.int32),
            pltpu.VMEM((_K,), jnp.int32),
            pltpu.VMEM((_K,), jnp.float32),
            pltpu.VMEM((_K, d), jnp.float32),
            pltpu.SemaphoreType.DMA,
        ],
        compiler_params=pltpu.CompilerParams(use_tc_tiling_on_sc=False),
    )
    def ksum(vals_hbm, src_hbm, dst_hbm, w_hbm, zero_hbm, out_hbm,
             acc, src_v, dst_v, w_v, rows_v, gsem):
        c = lax.axis_index("c")
        s = lax.axis_index("s")
        wid = s * _NC + c
        ebase = wid * ept
        rbase = s * rpt_main

        @pl.when(s < _NS - 1)
        def _():
            pltpu.sync_copy(zero_hbm.at[pl.ds(rbase, rpt_main)],
                            acc.at[pl.ds(rbase, rpt_main)])

        @pl.when(s == _NS - 1)
        def _():
            pltpu.sync_copy(zero_hbm.at[pl.ds(rbase, rpt_last)],
                            acc.at[pl.ds(rbase, rpt_last)])

        plsc.subcore_barrier()

        def chunk(jc, carry):
            base = ebase + jc * _K
            pltpu.sync_copy(src_hbm.at[pl.ds(base, _K)], src_v)
            pltpu.sync_copy(dst_hbm.at[pl.ds(base, _K)], dst_v)
            pltpu.sync_copy(w_hbm.at[pl.ds(base, _K)], w_v)
            pltpu.async_copy(vals_hbm.at[src_v], rows_v, gsem).wait()

            def edge_grp(g, carry2):
                w16 = w_v[pl.ds(g * _LANES, _LANES)]
                for i in range(_LANES):
                    e = g * _LANES + i
                    wb = w16[i]
                    for db in range(d // _LANES):
                        sl = pl.ds(db * _LANES, _LANES)
                        rows_v[e, sl] = rows_v[e, sl] * wb
                return carry2

            lax.fori_loop(0, _K // _LANES, edge_grp, 0)
            pltpu.sync_copy(rows_v, acc.at[dst_v], add=True)
            return carry

        lax.fori_loop(0, nchunk, chunk, 0)
        plsc.subcore_barrier()

        @pl.when(s < _NS - 1)
        def _():
            pltpu.sync_copy(acc.at[pl.ds(rbase, rpt_main)],
                            out_hbm.at[pl.ds(c * n_nodes + rbase, rpt_main)])

        @pl.when(s == _NS - 1)
        def _():
            pltpu.sync_copy(acc.at[pl.ds(rbase, rpt_last)],
                            out_hbm.at[pl.ds(c * n_nodes + rbase, rpt_last)])

    zero = jnp.zeros((n_nodes, d), jnp.float32)
    return ksum(vals, src, dst, w, zero)


def _dense_tc(parts, layer1, layer2, mask1, mask2):
    n = parts.shape[1]
    in_d = layer1.shape[0]
    out_d = layer2.shape[1]
    t_steps = mask1.shape[0]
    bn = 2000

    def body(p_ref, w1_ref, w2_ref, m1_ref, m2_ref, o_ref):
        h0 = p_ref[0] + p_ref[1]
        cols = []
        for t in range(t_steps):
            w1 = w1_ref[...] * m1_ref[t]
            h = jnp.maximum(
                jnp.dot(h0, w1, preferred_element_type=jnp.float32,
                        precision=lax.Precision.HIGHEST), 0.0)
            w2 = w2_ref[...] * m2_ref[t]
            cols.append(jnp.dot(h, w2, preferred_element_type=jnp.float32,
                                precision=lax.Precision.HIGHEST))
        o_ref[...] = jnp.concatenate(cols, axis=1)

    return pl.pallas_call(
        body,
        grid=(n // bn,),
        in_specs=[
            pl.BlockSpec((_NC, bn, in_d), lambda i: (0, i, 0)),
            pl.BlockSpec(layer1.shape, lambda i: (0, 0)),
            pl.BlockSpec(layer2.shape, lambda i: (0, 0)),
            pl.BlockSpec(mask1.shape, lambda i: (0, 0, 0)),
            pl.BlockSpec(mask2.shape, lambda i: (0, 0, 0)),
        ],
        out_specs=pl.BlockSpec((bn, t_steps * out_d), lambda i: (i, 0)),
        out_shape=jax.ShapeDtypeStruct((n, t_steps * out_d), jnp.float32),
    )(parts, layer1, layer2, mask1, mask2)


def _finalize_tc(parts):
    _, n, td = parts.shape
    out_d = td // T
    bn = 2000

    def body(p_ref, o_ref):
        ssum = p_ref[0] + p_ref[1]
        for t in range(T):
            o_ref[t] = ssum[:, t * out_d:(t + 1) * out_d]

    return pl.pallas_call(
        body,
        grid=(n // bn,),
        in_specs=[pl.BlockSpec((_NC, bn, td), lambda i: (0, i, 0))],
        out_specs=pl.BlockSpec((T, bn, out_d), lambda i: (0, i, 0)),
        out_shape=jax.ShapeDtypeStruct((T, n, out_d), jnp.float32),
    )(parts)


def kernel(x, edge_index, edge_weight, layer1, layer2, mask1, mask2):
    n = x.shape[0]
    src = edge_index[0]
    dst = edge_index[1]
    part_a = _seg_sum_sc(x, src, dst, edge_weight)
    p = _dense_tc(part_a.reshape(_NC, n, x.shape[1]),
                  layer1, layer2, mask1, mask2)
    part_c = _seg_sum_sc(p, src, dst, edge_weight)
    return _finalize_tc(part_c.reshape(_NC, n, T * layer2.shape[1]))
